# baseline (speedup 1.0000x reference)
import functools

import jax
import jax.numpy as jnp
from jax import lax
from jax.experimental import pallas as pl
from jax.experimental.pallas import tpu as pltpu
from jax.experimental.pallas import tpu_sc as plsc

BATCH = 16384
VOCAB = 1000000
DEPTH = 64
NC = 2
NS = 16
NW = NC * NS
BPW = BATCH // NW
G = 128
NG = BPW // G

_mesh = plsc.VectorSubcoreMesh(core_axis_name="c", subcore_axis_name="s")


@functools.partial(
    pl.kernel,
    mesh=_mesh,
    out_type=jax.ShapeDtypeStruct((BATCH, 2 * DEPTH), jnp.float32),
    scratch_types=[
        pltpu.VMEM((NG, G), jnp.int32),
        pltpu.VMEM((BPW, 2 * DEPTH), jnp.float32),
        pltpu.SemaphoreType.DMA,
    ],
)
def _gather_kernel(idx_hbm, table_hbm, out_hbm, idx_v, stage_v, sem):
    wid = lax.axis_index("s") * NC + lax.axis_index("c")
    pltpu.sync_copy(idx_hbm.at[wid], idx_v)
    copies = []
    for j in range(NG):
        copies.append(
            pltpu.async_copy(
                table_hbm.at[idx_v.at[j]], stage_v.at[pl.ds(j * G, G)], sem))
    for cp in copies:
        cp.wait()
    pltpu.sync_copy(stage_v, out_hbm.at[pl.ds(wid * BPW, BPW)])


def kernel(x, embedding):
    flat = jnp.reshape(x, (BATCH,))
    idx = jnp.reshape(flat >> 1, (NW, NG, G))
    packed = jnp.reshape(embedding, (VOCAB // 2, 2 * DEPTH))
    pairs = _gather_kernel(idx, packed)
    odd = (flat & 1)[:, None] == 1
    return jnp.where(odd, pairs[:, DEPTH:], pairs[:, :DEPTH])
